# baseline (speedup 1.0000x reference)
import functools

import jax
import jax.numpy as jnp
from jax import lax
from jax.experimental import pallas as pl
from jax.experimental.pallas import tpu as pltpu
from jax.experimental.pallas import tpu_sc as plsc

BATCH = 4096
SEQ = 26
DIM = 64
TOTAL = BATCH * SEQ
NROWS = 1000000
NUM_CORES = 2
NUM_SUBCORES = 16
NW = NUM_CORES * NUM_SUBCORES
ROWS_PER_W = TOTAL // NW
CHUNK = 128
N_CHUNKS = ROWS_PER_W // CHUNK
NBUF = 6
L = 16
TCOLS = 500


TBLK = 1024
TGRID = -(-NROWS // TBLK)
TROWS = TGRID * (TBLK // 2)


def _repack_body(x_ref, o_ref):
    x = x_ref[...]
    o_ref[...] = jnp.concatenate(
        [x[:, :TBLK // 2].T, x[:, TBLK // 2:].T], axis=1)


def _repack(weight):
    wt = weight.T
    return pl.pallas_call(
        _repack_body,
        grid=(TGRID,),
        in_specs=[pl.BlockSpec((DIM, TBLK), lambda i: (0, i))],
        out_specs=pl.BlockSpec((TBLK // 2, 2 * DIM), lambda i: (i, 0)),
        out_shape=jax.ShapeDtypeStruct((TROWS, 2 * DIM), jnp.float32),
    )(wt)


@functools.partial(
    pl.kernel,
    mesh=plsc.VectorSubcoreMesh(core_axis_name="c", subcore_axis_name="s"),
    out_type=jax.ShapeDtypeStruct((TOTAL // 2, 2 * DIM), jnp.float32),
    scratch_types=(
        [
            pltpu.VMEM((ROWS_PER_W,), jnp.int32),
            pltpu.VMEM((ROWS_PER_W,), jnp.int32),
            pltpu.VMEM((ROWS_PER_W,), jnp.int32),
        ]
        + [pltpu.VMEM((CHUNK, 2 * DIM), jnp.float32) for _ in range(NBUF)]
        + [pltpu.VMEM((CHUNK // 2, 2 * DIM), jnp.float32) for _ in range(2)]
        + [pltpu.SemaphoreType.DMA for _ in range(NBUF + 2)]
    ),
    compiler_params=pltpu.CompilerParams(use_tc_tiling_on_sc=False),
)
def _gather_rows(idx_hbm, table_hbm, out_hbm, idx_v, idx2_v, half_v,
                 *bufs_and_sems):
    bufs = bufs_and_sems[:NBUF]
    obufs = bufs_and_sems[NBUF:NBUF + 2]
    gsem = bufs_and_sems[NBUF + 2:2 * NBUF + 2]
    ssem = bufs_and_sems[2 * NBUF + 2:]
    cid = lax.axis_index("c")
    sid = lax.axis_index("s")
    wid = sid * NUM_CORES + cid
    base2 = wid * (ROWS_PER_W // 2)
    pltpu.sync_copy(idx_hbm.at[wid], idx_v)

    def _prep(g, _):
        off = g * L
        r = idx_v[pl.ds(off, L)]
        idx2_v[pl.ds(off, L)] = (
            lax.shift_left(lax.shift_right_logical(r, 10), 9)
            + lax.bitwise_and(r, 511))
        half_v[pl.ds(off, L)] = lax.bitwise_and(
            lax.shift_right_logical(r, 9), 1)
        return 0

    lax.fori_loop(0, ROWS_PER_W // L, _prep, 0)

    def _assemble(j, b, ob):
        buf = bufs[b]
        obuf = obufs[ob]

        def _group(g, _):
            h16 = half_v[pl.ds(j * CHUNK + g * L, L)]
            for u in range(L):
                k = g * L + u
                src = h16[u] * DIM
                p = g * (L // 2) + u // 2
                c0 = (u % 2) * DIM
                for v in range(DIM // L):
                    obuf[p, pl.ds(c0 + v * L, L)] = (
                        buf[k, pl.ds(src + v * L, L)])
            return 0

        lax.fori_loop(0, CHUNK // L, _group, 0)

    gathers = [None] * N_CHUNKS
    for j in range(min(NBUF, N_CHUNKS)):
        gathers[j] = pltpu.async_copy(
            table_hbm.at[idx2_v.at[pl.ds(j * CHUNK, CHUNK)]],
            bufs[j % NBUF], gsem[j % NBUF])
    stores = [None] * N_CHUNKS
    for j in range(N_CHUNKS):
        b = j % NBUF
        ob = j % 2
        gathers[j].wait()
        if j >= 2:
            stores[j - 2].wait()
        _assemble(j, b, ob)
        stores[j] = pltpu.async_copy(
            obufs[ob], out_hbm.at[pl.ds(base2 + j * (CHUNK // 2), CHUNK // 2)],
            ssem[ob])
        nj = j + NBUF
        if nj < N_CHUNKS:
            gathers[nj] = pltpu.async_copy(
                table_hbm.at[idx2_v.at[pl.ds(nj * CHUNK, CHUNK)]],
                bufs[b], gsem[b])
    for j in (N_CHUNKS - 2, N_CHUNKS - 1):
        stores[j].wait()


def kernel(input_indices, weight):
    idx = input_indices.reshape(NW, ROWS_PER_W).astype(jnp.int32)
    w2 = _repack(weight)
    out = _gather_rows(idx, w2)
    return out.reshape(BATCH, SEQ, DIM)
